# Initial kernel scaffold; baseline (speedup 1.0000x reference)
import jax
import jax.numpy as jnp
from jax.experimental import pallas as pl


def kernel(x, edge_index, W):
    raise NotImplementedError("write your pallas kernel here")


# speedup vs baseline: 17.5025x; 17.5025x over previous
import functools

import jax
import jax.numpy as jnp
from jax import lax
from jax.experimental import pallas as pl
from jax.experimental.pallas import tpu as pltpu
from jax.experimental.pallas import tpu_sc as plsc

N = 10000
E = 320000
D = 128

NC = 2
NS = 16
NW = NC * NS
CHUNK = 128
CH = 79
EP = NW * CH * CHUNK
NPAD = 10240
ROWS_PER_TILE = NPAD // NS
BLK = 512
GRID = NPAD // BLK


def _mesh():
    return plsc.VectorSubcoreMesh(core_axis_name="c", subcore_axis_name="s")


def _deg_body(colp_hbm, out_hbm, col_v, ones_v, zbuf_v, dacc):
    c = lax.axis_index("c")
    s = lax.axis_index("s")
    w = c * NS + s

    @pl.loop(0, ROWS_PER_TILE // 16)
    def _(i):
        zbuf_v[pl.ds(i * 16, 16)] = jnp.zeros((16,), jnp.float32)

    for k in range(CHUNK // 16):
        ones_v[pl.ds(k * 16, 16)] = jnp.ones((16,), jnp.float32)

    pltpu.sync_copy(zbuf_v, dacc.at[pl.ds(s * ROWS_PER_TILE, ROWS_PER_TILE)])
    plsc.subcore_barrier()

    pltpu.sync_copy(colp_hbm.at[w], col_v)

    @pl.loop(0, CH)
    def _(j):
        pltpu.sync_copy(ones_v, dacc.at[col_v.at[j]], add=True)

    plsc.subcore_barrier()
    pltpu.sync_copy(
        dacc.at[pl.ds(s * ROWS_PER_TILE, ROWS_PER_TILE)],
        out_hbm.at[c, pl.ds(s * ROWS_PER_TILE, ROWS_PER_TILE)],
    )


@jax.jit
def _deg_sc(colp):
    return pl.kernel(
        _deg_body,
        out_type=jax.ShapeDtypeStruct((NC, NPAD), jnp.float32),
        mesh=_mesh(),
        scratch_types=[
            pltpu.VMEM((CH, CHUNK), jnp.int32),
            pltpu.VMEM((CHUNK,), jnp.float32),
            pltpu.VMEM((ROWS_PER_TILE,), jnp.float32),
            pltpu.VMEM_SHARED((NPAD,), jnp.float32),
        ],
    )(colp)


def _agg_body(xws_hbm, rowp_hbm, colp_hbm, out_hbm, row_v, col_v, rows_v, sem, acc):
    c = lax.axis_index("c")
    s = lax.axis_index("s")
    w = c * NS + s

    @pl.loop(0, CHUNK)
    def _(r):
        for k in range(D // 16):
            rows_v[r, pl.ds(k * 16, 16)] = jnp.zeros((16,), jnp.float32)

    for b in range(ROWS_PER_TILE // CHUNK):
        pltpu.sync_copy(
            rows_v, acc.at[pl.ds(s * ROWS_PER_TILE + b * CHUNK, CHUNK)]
        )
    plsc.subcore_barrier()

    pltpu.sync_copy(rowp_hbm.at[w], row_v)
    pltpu.sync_copy(colp_hbm.at[w], col_v)

    @pl.loop(0, CH)
    def _(j):
        pltpu.async_copy(xws_hbm.at[row_v.at[j]], rows_v, sem).wait()
        pltpu.sync_copy(rows_v, acc.at[col_v.at[j]], add=True)

    plsc.subcore_barrier()
    pltpu.sync_copy(
        acc.at[pl.ds(s * ROWS_PER_TILE, ROWS_PER_TILE)],
        out_hbm.at[c, pl.ds(s * ROWS_PER_TILE, ROWS_PER_TILE)],
    )


@jax.jit
def _agg_sc(xws, rowp, colp):
    return pl.kernel(
        _agg_body,
        out_type=jax.ShapeDtypeStruct((NC, NPAD, D), jnp.float32),
        mesh=_mesh(),
        scratch_types=[
            pltpu.VMEM((CH, CHUNK), jnp.int32),
            pltpu.VMEM((CH, CHUNK), jnp.int32),
            pltpu.VMEM((CHUNK, D), jnp.float32),
            pltpu.SemaphoreType.DMA,
            pltpu.VMEM_SHARED((NPAD, D), jnp.float32),
        ],
    )(xws, rowp, colp)


def _mm_body(x_ref, w_ref, d0_ref, d1_ref, o_ref):
    d = d0_ref[...] + d1_ref[...]
    dinv = jnp.where(d > 0.0, lax.rsqrt(d), 0.0)
    xw = jnp.dot(x_ref[...], w_ref[...], preferred_element_type=jnp.float32)
    o_ref[...] = xw * dinv[:, None]


@jax.jit
def _mm_tc(xp, W, d0, d1):
    return pl.pallas_call(
        _mm_body,
        grid=(GRID,),
        in_specs=[
            pl.BlockSpec((BLK, D), lambda i: (i, 0)),
            pl.BlockSpec((D, D), lambda i: (0, 0)),
            pl.BlockSpec((BLK,), lambda i: (i,)),
            pl.BlockSpec((BLK,), lambda i: (i,)),
        ],
        out_specs=pl.BlockSpec((BLK, D), lambda i: (i, 0)),
        out_shape=jax.ShapeDtypeStruct((NPAD, D), jnp.float32),
    )(xp, W, d0, d1)


def _fin_body(p0_ref, p1_ref, d0_ref, d1_ref, o_ref):
    d = d0_ref[...] + d1_ref[...]
    dinv = jnp.where(d > 0.0, lax.rsqrt(d), 0.0)
    o_ref[...] = (p0_ref[...] + p1_ref[...]) * dinv[:, None]


@jax.jit
def _fin_tc(p0, p1, d0, d1):
    return pl.pallas_call(
        _fin_body,
        grid=(GRID,),
        in_specs=[
            pl.BlockSpec((BLK, D), lambda i: (i, 0)),
            pl.BlockSpec((BLK, D), lambda i: (i, 0)),
            pl.BlockSpec((BLK,), lambda i: (i,)),
            pl.BlockSpec((BLK,), lambda i: (i,)),
        ],
        out_specs=pl.BlockSpec((BLK, D), lambda i: (i, 0)),
        out_shape=jax.ShapeDtypeStruct((NPAD, D), jnp.float32),
    )(p0, p1, d0, d1)


def kernel(x, edge_index, W):
    row = edge_index[0].astype(jnp.int32)
    col = edge_index[1].astype(jnp.int32)
    rowp = jnp.pad(row, (0, EP - E)).reshape(NW, CH, CHUNK)
    colp = jnp.pad(col, (0, EP - E), constant_values=N).reshape(NW, CH, CHUNK)

    deg2 = _deg_sc(colp)
    d0 = deg2[0]
    d1 = deg2[1]

    xp = jnp.pad(x, ((0, NPAD - N), (0, 0)))
    xws = _mm_tc(xp, W, d0, d1)

    part = _agg_sc(xws, rowp, colp)
    out = _fin_tc(part[0], part[1], d0, d1)
    return out[:N]
